# Initial kernel scaffold; baseline (speedup 1.0000x reference)
import functools

import jax
import jax.numpy as jnp
from jax import lax
from jax.experimental import pallas as pl
from jax.experimental.pallas import tpu as pltpu
from jax.experimental.pallas import tpu_sc as plsc

B, S = 4096, 200
VOCAB, D = 100000, 128
SEG = 2
L = 16
NC, NS = 2, 16
NW = NC * NS
N = B * S
ROWS_PER_W = N // NW
CHUNK = 128
N_CHUNKS = ROWS_PER_W // CHUNK

P_PER_W = 7
P_CLAMP = S - P_PER_W

_mesh = plsc.VectorSubcoreMesh(
    core_axis_name="c", subcore_axis_name="s", num_cores=NC, num_subcores=NS
)


@functools.partial(
    pl.kernel,
    out_type=jax.ShapeDtypeStruct((2 * S, D), jnp.float32),
    mesh=_mesh,
    scratch_types=[
        pltpu.VMEM((P_PER_W, D), jnp.float32),
        pltpu.VMEM((SEG, D), jnp.float32),
        pltpu.VMEM((2 * P_PER_W, D), jnp.float32),
    ],
)
def _build_comb(pos_hbm, seg_hbm, comb_hbm, pos_v, seg_v, comb_v):
    wid = lax.axis_index("s") * NC + lax.axis_index("c")
    p0 = jnp.minimum(wid * P_PER_W, P_CLAMP)
    pltpu.sync_copy(pos_hbm.at[pl.ds(p0, P_PER_W)], pos_v)
    pltpu.sync_copy(seg_hbm, seg_v)
    for pp in range(P_PER_W):
        for g in range(SEG):
            for j in range(D // L):
                sl = pl.ds(j * L, L)
                comb_v[2 * pp + g, sl] = pos_v[pp, sl] + seg_v[g, sl]
    pltpu.sync_copy(comb_v, comb_hbm.at[pl.ds(2 * p0, 2 * P_PER_W)])


@functools.partial(
    pl.kernel,
    out_type=jax.ShapeDtypeStruct((N, D), jnp.float32),
    mesh=_mesh,
    scratch_types=[
        pltpu.VMEM((CHUNK,), jnp.int32),
        pltpu.VMEM((CHUNK,), jnp.int32),
        pltpu.VMEM((CHUNK,), jnp.int32),
        pltpu.VMEM((CHUNK, D), jnp.float32),
        pltpu.VMEM((CHUNK, D), jnp.float32),
        pltpu.SemaphoreType.DMA,
        pltpu.SemaphoreType.DMA,
    ],
)
def _embed(ids_hbm, seg_hbm, tok_tab_hbm, comb_hbm, out_hbm,
           idx_v, seg_v, cidx_v, tok_v, comb_v, sem_tok, sem_comb):
    wid = lax.axis_index("s") * NC + lax.axis_index("c")
    base0 = wid * ROWS_PER_W
    lane = lax.iota(jnp.int32, L)

    def chunk_body(i, carry):
        base = base0 + i * CHUNK
        pltpu.sync_copy(ids_hbm.at[pl.ds(base, CHUNK)], idx_v)
        pltpu.sync_copy(seg_hbm.at[pl.ds(base, CHUNK)], seg_v)
        tok_cp = pltpu.async_copy(tok_tab_hbm.at[idx_v], tok_v, sem_tok)
        for m in range(CHUNK // L):
            sl = pl.ds(m * L, L)
            flat = lane + (base + m * L)
            cidx_v[sl] = lax.rem(flat, S) * 2 + seg_v[sl]
        comb_cp = pltpu.async_copy(comb_hbm.at[cidx_v], comb_v, sem_comb)
        tok_cp.wait()
        comb_cp.wait()

        def add_row(r, inner):
            for j in range(D // L):
                sl = pl.ds(j * L, L)
                plsc.addupdate(tok_v.at[r, sl], comb_v[r, sl])
            return inner

        lax.fori_loop(0, CHUNK, add_row, 0)
        pltpu.sync_copy(tok_v, out_hbm.at[pl.ds(base, CHUNK)])
        return carry

    lax.fori_loop(0, N_CHUNKS, chunk_body, 0)


def kernel(input_ids, segment_ids, token_table, position_table, segment_table):
    ids_flat = input_ids.reshape(N).astype(jnp.int32)
    seg_flat = segment_ids.reshape(N).astype(jnp.int32)
    comb = _build_comb(position_table[:S], segment_table)
    out = _embed(ids_flat, seg_flat, token_table, comb)
    return out.reshape(B, S, D)


# speedup vs baseline: 8.8497x; 8.8497x over previous
import functools

import jax
import jax.numpy as jnp
from jax import lax
from jax.experimental import pallas as pl
from jax.experimental.pallas import tpu as pltpu
from jax.experimental.pallas import tpu_sc as plsc

B, S = 4096, 200
VOCAB, D = 100000, 128
SEG = 2
L = 16
NC, NS = 2, 16
NW = NC * NS
N = B * S
ROWS_PER_W = N // NW
CHUNK = 128
N_CHUNKS = ROWS_PER_W // CHUNK

P_PER_W = 8
P_CLAMP = S - P_PER_W

_mesh = plsc.VectorSubcoreMesh(
    core_axis_name="c", subcore_axis_name="s", num_cores=NC, num_subcores=NS
)


@functools.partial(
    pl.kernel,
    out_type=jax.ShapeDtypeStruct((2 * S, D), jnp.float32),
    mesh=_mesh,
    scratch_types=[
        pltpu.VMEM((P_PER_W, D), jnp.float32),
        pltpu.VMEM((SEG, D), jnp.float32),
        pltpu.VMEM((2 * P_PER_W, D), jnp.float32),
    ],
)
def _build_comb(pos_hbm, seg_hbm, comb_hbm, pos_v, seg_v, comb_v):
    wid = lax.axis_index("s") * NC + lax.axis_index("c")
    p0 = jnp.minimum(wid * P_PER_W, P_CLAMP)
    pltpu.sync_copy(pos_hbm.at[pl.ds(p0, P_PER_W)], pos_v)
    pltpu.sync_copy(seg_hbm, seg_v)
    for pp in range(P_PER_W):
        for g in range(SEG):
            for j in range(D // L):
                sl = pl.ds(j * L, L)
                comb_v[2 * pp + g, sl] = pos_v[pp, sl] + seg_v[g, sl]
    pltpu.sync_copy(comb_v, comb_hbm.at[pl.ds(2 * p0, 2 * P_PER_W)])


@functools.partial(
    pl.kernel,
    out_type=jax.ShapeDtypeStruct((N, D), jnp.float32),
    mesh=_mesh,
    scratch_types=[
        pltpu.VMEM((CHUNK,), jnp.int32),
        pltpu.VMEM((CHUNK,), jnp.int32),
        pltpu.VMEM((CHUNK,), jnp.int32),
        pltpu.VMEM((CHUNK, D), jnp.float32),
        pltpu.VMEM((CHUNK, D), jnp.float32),
        pltpu.SemaphoreType.DMA,
        pltpu.SemaphoreType.DMA,
    ],
)
def _embed(ids_hbm, seg_hbm, tok_tab_hbm, comb_hbm, out_hbm,
           idx_v, seg_v, cidx_v, tok_v, comb_v, sem_tok, sem_comb):
    wid = lax.axis_index("s") * NC + lax.axis_index("c")
    base0 = wid * ROWS_PER_W
    lane = lax.iota(jnp.int32, L)

    def chunk_body(i, carry):
        base = base0 + i * CHUNK
        pltpu.sync_copy(ids_hbm.at[pl.ds(base, CHUNK)], idx_v)
        pltpu.sync_copy(seg_hbm.at[pl.ds(base, CHUNK)], seg_v)
        tok_cp = pltpu.async_copy(tok_tab_hbm.at[idx_v], tok_v, sem_tok)
        for m in range(CHUNK // L):
            sl = pl.ds(m * L, L)
            flat = lane + (base + m * L)
            cidx_v[sl] = lax.rem(flat, S) * 2 + seg_v[sl]
        comb_cp = pltpu.async_copy(comb_hbm.at[cidx_v], comb_v, sem_comb)
        tok_cp.wait()
        comb_cp.wait()

        def add_row(r, inner):
            for j in range(D // L):
                sl = pl.ds(j * L, L)
                plsc.addupdate(tok_v.at[r, sl], comb_v[r, sl])
            return inner

        lax.fori_loop(0, CHUNK, add_row, 0)
        pltpu.sync_copy(tok_v, out_hbm.at[pl.ds(base, CHUNK)])
        return carry

    lax.fori_loop(0, N_CHUNKS, chunk_body, 0)


def kernel(input_ids, segment_ids, token_table, position_table, segment_table):
    ids_flat = input_ids.reshape(N).astype(jnp.int32)
    seg_flat = segment_ids.reshape(N).astype(jnp.int32)
    comb = _build_comb(position_table[:S], segment_table)
    out = _embed(ids_flat, seg_flat, token_table, comb)
    return out.reshape(B, S, D)
